# baseline (speedup 1.0000x reference)
import functools

import jax
import jax.numpy as jnp
import numpy as np
from jax import lax
from jax.experimental import pallas as pl
from jax.experimental.pallas import tpu as pltpu
from jax.experimental.pallas import tpu_sc as plsc

NC = 2
NS = 16
NW = NC * NS
LANES = 16

ROWS = 128
COLS = 32768
SC_ROWS = 64
ROWS_PER_W = SC_ROWS // NW

ACC = 4
UNROLL = 4
CHUNK = 16384
VREGS_PER_STEP = ACC * UNROLL
STEPS_PER_CHUNK = CHUNK // (VREGS_PER_STEP * LANES)

_INT_MAX = np.int32(2147483647)


@functools.partial(
    pl.kernel,
    mesh=plsc.VectorSubcoreMesh(core_axis_name="c", subcore_axis_name="s"),
    out_type=jax.ShapeDtypeStruct((NW, LANES), jnp.int32),
    compiler_params=pltpu.CompilerParams(needs_layout_passes=False),
    scratch_types=[
        pltpu.VMEM((CHUNK,), jnp.float32),
        pltpu.VMEM((CHUNK,), jnp.float32),
        pltpu.VMEM((LANES,), jnp.int32),
        pltpu.SemaphoreType.DMA,
        pltpu.SemaphoreType.DMA,
    ],
)
def _argmax_sc(x_hbm, out_hbm, buf0, buf1, res_ref, sem0, sem1):
    bufs = (buf0, buf1)
    sems = (sem0, sem1)
    wid = lax.axis_index("s") * NC + lax.axis_index("c")
    base_row = wid * ROWS_PER_W
    lane = lax.iota(jnp.int32, LANES)

    def chunk_src(c):
        return x_hbm.at[base_row + c // 2, pl.ds((c % 2) * CHUNK, CHUNK)]

    pltpu.async_copy(chunk_src(0), buf0, sem0)
    pltpu.async_copy(chunk_src(1), buf1, sem1)

    neg_inf = jnp.full((LANES,), -jnp.inf, jnp.float32)
    zero_i = jnp.zeros((LANES,), jnp.int32)

    def row_body(j, res):
        best = [neg_inf] * ACC
        bidx = [zero_i] * ACC
        idx = [lane + a * LANES for a in range(ACC)]

        for half in range(2):
            c = 2 * j + half
            buf = bufs[half]
            pltpu.make_async_copy(chunk_src(c), buf, sems[half]).wait()

            def step(i, carry, buf=buf):
                best, bidx, idx = list(carry[0]), list(carry[1]), list(carry[2])
                for u in range(UNROLL):
                    for a in range(ACC):
                        k = i * VREGS_PER_STEP + u * ACC + a
                        v = buf[pl.ds(k * LANES, LANES)]
                        m = v > best[a]
                        best[a] = jnp.where(m, v, best[a])
                        bidx[a] = jnp.where(m, idx[a], bidx[a])
                        idx[a] = idx[a] + ACC * LANES
                return tuple(best), tuple(bidx), tuple(idx)

            carry = lax.fori_loop(
                0, STEPS_PER_CHUNK, step, (tuple(best), tuple(bidx), tuple(idx))
            )
            best, bidx, idx = list(carry[0]), list(carry[1]), list(carry[2])

            @pl.when(j < ROWS_PER_W - 1)
            def _issue(c=c, half=half):
                pltpu.async_copy(chunk_src(c + 2), bufs[half], sems[half])

        def merge(b1, i1, b2, i2):
            m = (b2 > b1) | ((b2 == b1) & (i2 < i1))
            return jnp.where(m, b2, b1), jnp.where(m, i2, i1)

        b01, i01 = merge(best[0], bidx[0], best[1], bidx[1])
        b23, i23 = merge(best[2], bidx[2], best[3], bidx[3])
        ball, iall = merge(b01, i01, b23, i23)

        row_max = jnp.max(ball)
        cand = jnp.where(ball == row_max, iall, _INT_MAX)
        row_arg = jnp.min(cand)
        return jnp.where(lane == j, row_arg, res)

    res = lax.fori_loop(0, ROWS_PER_W, row_body, jnp.zeros((LANES,), jnp.int32))

    res_ref[...] = res
    pltpu.sync_copy(res_ref, out_hbm.at[wid])


RB = 8
W = 128
NT = COLS // W
TC_ROWS = ROWS - SC_ROWS
TC_BLOCK_OFF = SC_ROWS // RB


def _tc_body(x_ref, o_ref):
    xs = [x_ref[:, pl.ds(t * W, W)] for t in range(NT)]
    vals = xs
    while len(vals) > 1:
        vals = [jnp.maximum(a, b) for a, b in zip(vals[::2], vals[1::2])]
    rowmax = jnp.max(vals[0], axis=1, keepdims=True)
    col0 = jax.lax.broadcasted_iota(jnp.int32, (RB, W), 1)
    cands = [
        jnp.where(xs[t] == rowmax, col0 + t * W, _INT_MAX) for t in range(NT)
    ]
    while len(cands) > 1:
        cands = [jnp.minimum(a, b) for a, b in zip(cands[::2], cands[1::2])]
    o_ref[...] = jnp.min(cands[0], axis=1).reshape(1, 1, RB)


def _argmax_tc(x):
    return pl.pallas_call(
        _tc_body,
        grid=(TC_ROWS // RB,),
        in_specs=[pl.BlockSpec((RB, COLS), lambda i: (i + TC_BLOCK_OFF, 0))],
        out_specs=pl.BlockSpec((1, 1, RB), lambda i: (i, 0, 0)),
        out_shape=jax.ShapeDtypeStruct((TC_ROWS // RB, 1, RB), jnp.int32),
        compiler_params=pltpu.CompilerParams(
            dimension_semantics=("arbitrary",),
        ),
    )(x)


def kernel(x):
    sc_out = _argmax_sc(x)
    tc_out = _argmax_tc(x)
    sc_part = sc_out[:, :ROWS_PER_W].reshape(SC_ROWS)
    tc_part = tc_out.reshape(TC_ROWS)
    return jnp.concatenate([sc_part, tc_part]).astype(jnp.int64)
